# Initial kernel scaffold; baseline (speedup 1.0000x reference)
import jax
import jax.numpy as jnp
from jax.experimental import pallas as pl


def kernel(obj_label, qus_type, attention, score_matrix):
    raise NotImplementedError("write your pallas kernel here")


# speedup vs baseline: 1.3485x; 1.3485x over previous
import functools

import jax
import jax.numpy as jnp
from jax import lax
from jax.experimental import pallas as pl
from jax.experimental.pallas import tpu as pltpu
from jax.experimental.pallas import tpu_sc as plsc

NUM_QT = 65
NUM_OT = 151
PAIR_NUM = 90
BOX = 10
ROWS = NUM_QT * PAIR_NUM
CHUNK = 256

NC, NS = 2, 16
NW = NC * NS


def _sc_body(score_hbm, obj_hbm, att_hbm, starts_hbm, out_hbm,
             slab, obj_buf, att_buf, starts_v):
    w = lax.axis_index("s") * NC + lax.axis_index("c")
    pltpu.sync_copy(starts_hbm, starts_v)
    g_lo = (w * ROWS) // NW
    g_hi = ((w + 1) * ROWS) // NW

    def group_body(g, carry):
        qt = g // PAIR_NUM
        ra = g - qt * PAIR_NUM
        i = ra // (BOX - 1)
        r = ra - i * (BOX - 1)
        j = jnp.where(r >= i, r + 1, r)
        sv = starts_v[pl.ds(qt, 16)]
        base = sv[0]
        cnt = sv[1] - sv[0]
        aligned = (base // 8) * 8
        off = base - aligned
        pltpu.sync_copy(score_hbm.at[qt, ra], slab)

        def chunk_body(ch, carry2):
            rowbase = aligned + ch * CHUNK
            pltpu.sync_copy(obj_hbm.at[pl.ds(rowbase, CHUNK)], obj_buf)
            pltpu.sync_copy(att_hbm.at[pl.ds(rowbase, CHUNK)], att_buf)
            n = jnp.minimum(off + cnt - ch * CHUNK, CHUNK)

            def vec_body(k, carry3):
                lane = lax.iota(jnp.int32, 16) + k * 16
                row = rowbase + lane
                m = (row >= base) & (row < base + cnt)
                ci = jnp.full((16,), i, jnp.int32)
                cj = jnp.full((16,), j, jnp.int32)
                o1 = plsc.load_gather(obj_buf, [lane, cj], mask=m)
                o2 = plsc.load_gather(obj_buf, [lane, ci], mask=m)
                a1 = plsc.load_gather(att_buf, [lane, cj], mask=m)
                a2 = plsc.load_gather(att_buf, [lane, ci], mask=m)
                val = a1 * a2
                plsc.addupdate_scatter(slab, [o1, o2], val, mask=m)
                return carry3

            lax.fori_loop(0, (n + 15) // 16, vec_body, 0)
            return carry2

        lax.fori_loop(0, (off + cnt + CHUNK - 1) // CHUNK, chunk_body, 0)
        pltpu.sync_copy(slab, out_hbm.at[qt, ra])
        return carry

    lax.fori_loop(g_lo, g_hi, group_body, 0)


@jax.jit
def kernel(obj_label, qus_type, attention, score_matrix):
    B = qus_type.shape[0]
    order = jnp.argsort(qus_type, stable=True)
    sorted_qt = qus_type[order]
    sorted_obj = obj_label[order]
    sorted_att = attention[order]
    starts = jnp.searchsorted(sorted_qt, jnp.arange(NUM_QT + 1, dtype=jnp.int32),
                              side="left").astype(jnp.int32)
    starts_pad = jnp.zeros((80,), jnp.int32).at[: NUM_QT + 1].set(starts)
    sorted_obj = jnp.concatenate(
        [sorted_obj, jnp.zeros((CHUNK, BOX), jnp.int32)], axis=0)
    sorted_att = jnp.concatenate(
        [sorted_att, jnp.zeros((CHUNK, BOX), jnp.float32)], axis=0)

    mesh = plsc.VectorSubcoreMesh(core_axis_name="c", subcore_axis_name="s")
    out = pl.kernel(
        _sc_body,
        out_type=jax.ShapeDtypeStruct(score_matrix.shape, jnp.float32),
        mesh=mesh,
        compiler_params=pltpu.CompilerParams(
            use_tc_tiling_on_sc=False, needs_layout_passes=False),
        scratch_types=[
            pltpu.VMEM((NUM_OT, NUM_OT), jnp.float32),
            pltpu.VMEM((CHUNK, BOX), jnp.int32),
            pltpu.VMEM((CHUNK, BOX), jnp.float32),
            pltpu.VMEM((80,), jnp.int32),
        ],
    )(score_matrix, sorted_obj, sorted_att, starts_pad)
    return out


# speedup vs baseline: 1.4123x; 1.0473x over previous
import functools

import jax
import jax.numpy as jnp
from jax import lax
from jax.experimental import pallas as pl
from jax.experimental.pallas import tpu as pltpu
from jax.experimental.pallas import tpu_sc as plsc

NUM_QT = 65
NUM_OT = 151
PAIR_NUM = 90
BOX = 10
B = 16384
ROWS = NUM_QT * PAIR_NUM
CHUNK = 384
NSUB = CHUNK // 128
IDXN = 17440

NC, NS = 2, 16
NW = NC * NS


def _sc_body(score_hbm, data_hbm, qtyp_hbm, out_hbm,
             slab, rows_c, idx_buf, idx_stage, qtyp_v, sem):
    w = lax.axis_index("s") * NC + lax.axis_index("c")
    g_lo = (w * ROWS) // NW
    g_hi = ((w + 1) * ROWS) // NW
    qt0 = g_lo // PAIR_NUM
    qtN = (g_hi - 1) // PAIR_NUM

    pltpu.sync_copy(qtyp_hbm, qtyp_v)

    def zero_body(k, c):
        idx_buf[pl.ds(k * 16, 16)] = jnp.zeros((16,), jnp.int32)
        return c
    lax.fori_loop(0, IDXN // 16, zero_body, 0)

    offs = []
    cnts = []
    cursor = jnp.int32(0)
    for t in range(4):
        qt_t = qt0 + t
        valid_t = qt_t <= qtN

        def scan_body(k, cur):
            ids = lax.iota(jnp.int32, 16) + k * 16
            v = qtyp_v[pl.ds(k * 16, 16)]
            m = (v == qt_t) & valid_t
            mi = m.astype(jnp.int32)
            incl = plsc.cumsum(mi)
            pos = cur + incl - mi
            plsc.store_scatter(idx_buf, [pos], ids, mask=m)
            return cur + incl[15]

        new_cursor = lax.fori_loop(0, B // 16, scan_body, cursor)
        offs.append(cursor)
        cnts.append(new_cursor - cursor)
        cursor = ((new_cursor + 15) // 16) * 16

    def gather_rows(basex, ch):
        def sub(d, c):
            def stage(k, c2):
                idx_stage[d, pl.ds(k * 16, 16)] = idx_buf[
                    pl.ds(basex + ch * CHUNK + d * 128 + k * 16, 16)]
                return c2
            lax.fori_loop(0, 8, stage, 0)
            pltpu.async_copy(data_hbm.at[idx_stage.at[d]],
                             rows_c.at[pl.ds(d * 128, 128)], sem).wait()
            return c
        lax.fori_loop(0, NSUB, sub, 0)

    def group_body(g, carry):
        qt = g // PAIR_NUM
        ra = g - qt * PAIR_NUM
        i = ra // (BOX - 1)
        r = ra - i * (BOX - 1)
        j = jnp.where(r >= i, r + 1, r)
        t = qt - qt0
        base = offs[0]
        cnt = cnts[0]
        for tt in (1, 2, 3):
            base = jnp.where(t == tt, offs[tt], base)
            cnt = jnp.where(t == tt, cnts[tt], cnt)
        nch = (cnt + CHUNK - 1) // CHUNK

        pltpu.sync_copy(score_hbm.at[qt, ra], slab)

        def process_chunk(ch):
            n = jnp.minimum(cnt - ch * CHUNK, CHUNK)

            def vec_body(k, c3):
                lane = lax.iota(jnp.int32, 16) + k * 16
                m = lane < n
                ci = jnp.full((16,), i, jnp.int32)
                cj = jnp.full((16,), j, jnp.int32)
                o1 = plsc.load_gather(rows_c, [lane, cj], mask=m)
                o2 = plsc.load_gather(rows_c, [lane, ci], mask=m)
                a1 = plsc.bitcast(
                    plsc.load_gather(rows_c, [lane, cj + 10], mask=m),
                    jnp.float32)
                a2 = plsc.bitcast(
                    plsc.load_gather(rows_c, [lane, ci + 10], mask=m),
                    jnp.float32)
                plsc.addupdate_scatter(slab, [o1, o2], a1 * a2, mask=m)
                return c3

            lax.fori_loop(0, (n + 15) // 16, vec_body, 0)

        @pl.when((ra == 0) | (g == g_lo) | (nch > 1))
        def _():
            gather_rows(base, 0)

        process_chunk(0)

        def extra_chunk(ch, c2):
            gather_rows(base, ch)
            process_chunk(ch)
            return c2

        lax.fori_loop(1, nch, extra_chunk, 0)
        pltpu.sync_copy(slab, out_hbm.at[qt, ra])
        return carry

    lax.fori_loop(g_lo, g_hi, group_body, 0)


@jax.jit
def kernel(obj_label, qus_type, attention, score_matrix):
    data = jnp.zeros((B, 128), jnp.int32)
    data = data.at[:, :BOX].set(obj_label)
    data = data.at[:, BOX:2 * BOX].set(
        lax.bitcast_convert_type(attention, jnp.int32))

    mesh = plsc.VectorSubcoreMesh(core_axis_name="c", subcore_axis_name="s")
    out = pl.kernel(
        _sc_body,
        out_type=jax.ShapeDtypeStruct(score_matrix.shape, jnp.float32),
        mesh=mesh,
        compiler_params=pltpu.CompilerParams(
            use_tc_tiling_on_sc=False, needs_layout_passes=False),
        scratch_types=[
            pltpu.VMEM((NUM_OT, NUM_OT), jnp.float32),
            pltpu.VMEM((CHUNK, 128), jnp.int32),
            pltpu.VMEM((IDXN,), jnp.int32),
            pltpu.VMEM((8, 128), jnp.int32),
            pltpu.VMEM((B,), jnp.int32),
            pltpu.SemaphoreType.DMA,
        ],
    )(score_matrix, data, qus_type)
    return out


# speedup vs baseline: 10.0043x; 7.0838x over previous
import functools

import jax
import jax.numpy as jnp
from jax import lax
from jax.experimental import pallas as pl
from jax.experimental.pallas import tpu as pltpu
from jax.experimental.pallas import tpu_sc as plsc

NUM_QT = 65
NUM_OT = 151
PAIR_NUM = 90
BOX = 10
B = 16384
NPLANES = NUM_QT * NUM_OT
CHUNK = 384
NSUB = CHUNK // 128
IDXN = 17440
ENTN = CHUNK * BOX + 16

NC, NS = 2, 16
NW = NC * NS


def _sc_body(score_hbm, data_hbm, qtyp_hbm, out_hbm,
             slab, rows_c, idx_buf, idx_stage, qtyp_v,
             entries, cnts_b, bases_b, fill_b, sem):
    w = lax.axis_index("s") * NC + lax.axis_index("c")
    p_lo = (w * NPLANES) // NW
    p_hi = ((w + 1) * NPLANES) // NW
    qt0 = p_lo // NUM_OT
    qtN = (p_hi - 1) // NUM_OT

    pltpu.sync_copy(qtyp_hbm, qtyp_v)

    def zero_body(k, c):
        idx_buf[pl.ds(k * 16, 16)] = jnp.zeros((16,), jnp.int32)
        return c
    lax.fori_loop(0, IDXN // 16, zero_body, 0)

    offs = []
    cnts = []
    cursor = jnp.int32(0)
    for t in range(4):
        qt_t = qt0 + t
        valid_t = qt_t <= qtN

        def scan_body(k, cur):
            ids = lax.iota(jnp.int32, 16) + k * 16
            v = qtyp_v[pl.ds(k * 16, 16)]
            m = (v == qt_t) & valid_t
            mi = m.astype(jnp.int32)
            incl = plsc.cumsum(mi)
            pos = cur + incl - mi
            plsc.store_scatter(idx_buf, [pos], ids, mask=m)
            return cur + incl[15]

        new_cursor = lax.fori_loop(0, B // 16, scan_body, cursor)
        offs.append(cursor)
        cnts.append(new_cursor - cursor)
        cursor = ((new_cursor + 15) // 16) * 16

    def gather_rows(basex, ch):
        def sub(d, c):
            def stage(k, c2):
                idx_stage[d, pl.ds(k * 16, 16)] = idx_buf[
                    pl.ds(basex + ch * CHUNK + d * 128 + k * 16, 16)]
                return c2
            lax.fori_loop(0, 8, stage, 0)
            pltpu.async_copy(data_hbm.at[idx_stage.at[d]],
                             rows_c.at[pl.ds(d * 128, 128)], sem).wait()
            return c
        lax.fori_loop(0, NSUB, sub, 0)

    ones = jnp.ones((16,), jnp.int32)

    for t in range(4):
        qt = qt0 + t
        lo_t = jnp.clip(p_lo - qt * NUM_OT, 0, NUM_OT)
        hi_t = jnp.clip(p_hi - qt * NUM_OT, 0, NUM_OT)
        cnt_t = cnts[t]
        base_t = offs[t]
        nch = jnp.where(hi_t > lo_t,
                        jnp.maximum((cnt_t + CHUNK - 1) // CHUNK, 1), 0)

        def chunk_body(ch, c0, qt=qt, lo_t=lo_t, hi_t=hi_t,
                       cnt_t=cnt_t, base_t=base_t):
            gather_rows(base_t, ch)
            n = jnp.minimum(cnt_t - ch * CHUNK, CHUNK)
            nvec = jnp.maximum((n + 15) // 16, 0)

            for k in range(10):
                cnts_b[pl.ds(k * 16, 16)] = jnp.zeros((16,), jnp.int32)
                fill_b[pl.ds(k * 16, 16)] = jnp.zeros((16,), jnp.int32)

            for j in range(BOX):
                cj = jnp.full((16,), j, jnp.int32)

                def hist_body(k, c1, cj=cj):
                    lane = lax.iota(jnp.int32, 16) + k * 16
                    m = lane < n
                    o1 = plsc.load_gather(rows_c, [lane, cj], mask=m)
                    m2 = m & (o1 >= lo_t) & (o1 < hi_t)
                    plsc.addupdate_scatter(cnts_b, [o1 - lo_t], ones, mask=m2)
                    return c1
                lax.fori_loop(0, nvec, hist_body, 0)

            run = jnp.int32(0)
            for k in range(10):
                v = cnts_b[pl.ds(k * 16, 16)]
                incl = plsc.cumsum(v)
                bases_b[pl.ds(k * 16, 16)] = incl - v + run
                run = run + incl[15]

            for j in range(BOX):
                cj = jnp.full((16,), j, jnp.int32)

                def place_body(k, c1, cj=cj, j=j):
                    lane = lax.iota(jnp.int32, 16) + k * 16
                    m = lane < n
                    o1 = plsc.load_gather(rows_c, [lane, cj], mask=m)
                    m2 = m & (o1 >= lo_t) & (o1 < hi_t)
                    b = o1 - lo_t
                    cur = plsc.load_gather(fill_b, [b], mask=m2)
                    rank = plsc.scan_count(b, m2)[0] - 1
                    bv = plsc.load_gather(bases_b, [b], mask=m2)
                    pos = bv + cur + rank
                    plsc.store_scatter(entries, [pos], lane * 16 + j, mask=m2)
                    plsc.addupdate_scatter(fill_b, [b], ones, mask=m2)
                    return c1
                lax.fori_loop(0, nvec, place_body, 0)

            def plane_body(o1, c1, qt=qt, lo_t=lo_t):
                pltpu.sync_copy(score_hbm.at[qt, o1], slab)
                bsel = jnp.full((16,), o1 - lo_t, jnp.int32)
                eb = plsc.load_gather(bases_b, [bsel])[0]
                ecnt = plsc.load_gather(cnts_b, [bsel])[0]

                def ent_body(kk, c2):
                    lanez = lax.iota(jnp.int32, 16) + kk * 16
                    m3 = lanez < ecnt
                    ent = plsc.load_gather(entries, [eb + lanez], mask=m3)
                    row = ent // 16
                    jv = ent - row * 16
                    attj = plsc.bitcast(
                        plsc.load_gather(rows_c, [row, jv + 10], mask=m3),
                        jnp.float32)
                    for i in range(BOX):
                        ci = jnp.full((16,), i, jnp.int32)
                        m4 = m3 & (jv != i)
                        o2 = plsc.load_gather(rows_c, [row, ci], mask=m4)
                        atti = plsc.bitcast(
                            plsc.load_gather(rows_c, [row, ci + 10], mask=m4),
                            jnp.float32)
                        rav = i * 9 + jnp.where(jv < i, jv, jv - 1)
                        plsc.addupdate_scatter(
                            slab, [o2, rav], attj * atti, mask=m4)
                    return c2

                lax.fori_loop(0, (ecnt + 15) // 16, ent_body, 0)
                pltpu.sync_copy(slab, out_hbm.at[qt, o1])
                return c1

            lax.fori_loop(lo_t, hi_t, plane_body, 0)
            return c0

        lax.fori_loop(0, nch, chunk_body, 0)


@jax.jit
def kernel(obj_label, qus_type, attention, score_matrix):
    data = jnp.zeros((B, 128), jnp.int32)
    data = data.at[:, :BOX].set(obj_label)
    data = data.at[:, BOX:2 * BOX].set(
        lax.bitcast_convert_type(attention, jnp.int32))

    score_t = jnp.transpose(score_matrix, (0, 2, 3, 1))

    mesh = plsc.VectorSubcoreMesh(core_axis_name="c", subcore_axis_name="s")
    out_t = pl.kernel(
        _sc_body,
        out_type=jax.ShapeDtypeStruct(
            (NUM_QT, NUM_OT, NUM_OT, PAIR_NUM), jnp.float32),
        mesh=mesh,
        compiler_params=pltpu.CompilerParams(needs_layout_passes=False),
        scratch_types=[
            pltpu.VMEM((NUM_OT, PAIR_NUM), jnp.float32),
            pltpu.VMEM((CHUNK, 128), jnp.int32),
            pltpu.VMEM((IDXN,), jnp.int32),
            pltpu.VMEM((8, 128), jnp.int32),
            pltpu.VMEM((B,), jnp.int32),
            pltpu.VMEM((ENTN,), jnp.int32),
            pltpu.VMEM((160,), jnp.int32),
            pltpu.VMEM((160,), jnp.int32),
            pltpu.VMEM((160,), jnp.int32),
            pltpu.SemaphoreType.DMA,
        ],
    )(score_t, data, qus_type)
    return jnp.transpose(out_t, (0, 3, 1, 2))


# speedup vs baseline: 10.7171x; 1.0713x over previous
import functools

import jax
import jax.numpy as jnp
from jax import lax
from jax.experimental import pallas as pl
from jax.experimental.pallas import tpu as pltpu
from jax.experimental.pallas import tpu_sc as plsc

NUM_QT = 65
NUM_OT = 151
PAIR_NUM = 90
BOX = 10
B = 16384
NPLANES = NUM_QT * NUM_OT
CHUNK = 384
NSUB = CHUNK // 128
IDXN = 17440
ENTN = CHUNK * BOX + 16

NC, NS = 2, 16
NW = NC * NS


def _sc_body(score_hbm, data_hbm, qtyp_hbm, out_hbm,
             slab, slab_b, rows_c, idx_buf, idx_stage, qtyp_v,
             entries, cnts_b, bases_b, fill_b, sem, sem_a, sem_b):
    w = lax.axis_index("s") * NC + lax.axis_index("c")
    p_lo = (w * NPLANES) // NW
    p_hi = ((w + 1) * NPLANES) // NW
    qt0 = p_lo // NUM_OT
    qtN = (p_hi - 1) // NUM_OT

    pltpu.sync_copy(qtyp_hbm, qtyp_v)

    def zero_body(k, c):
        idx_buf[pl.ds(k * 16, 16)] = jnp.zeros((16,), jnp.int32)
        return c
    lax.fori_loop(0, IDXN // 16, zero_body, 0)

    offs = []
    cnts = []
    cursor = jnp.int32(0)
    for t in range(4):
        qt_t = qt0 + t
        valid_t = qt_t <= qtN

        def scan_body(k, cur):
            ids = lax.iota(jnp.int32, 16) + k * 16
            v = qtyp_v[pl.ds(k * 16, 16)]
            m = (v == qt_t) & valid_t
            mi = m.astype(jnp.int32)
            incl = plsc.cumsum(mi)
            pos = cur + incl - mi
            plsc.store_scatter(idx_buf, [pos], ids, mask=m)
            return cur + incl[15]

        new_cursor = lax.fori_loop(0, B // 16, scan_body, cursor)
        offs.append(cursor)
        cnts.append(new_cursor - cursor)
        cursor = ((new_cursor + 15) // 16) * 16

    def gather_rows(basex, ch):
        def sub(d, c):
            def stage(k, c2):
                idx_stage[d, pl.ds(k * 16, 16)] = idx_buf[
                    pl.ds(basex + ch * CHUNK + d * 128 + k * 16, 16)]
                return c2
            lax.fori_loop(0, 8, stage, 0)
            pltpu.async_copy(data_hbm.at[idx_stage.at[d]],
                             rows_c.at[pl.ds(d * 128, 128)], sem).wait()
            return c
        lax.fori_loop(0, NSUB, sub, 0)

    ones = jnp.ones((16,), jnp.int32)

    for t in range(4):
        qt = qt0 + t
        lo_t = jnp.clip(p_lo - qt * NUM_OT, 0, NUM_OT)
        hi_t = jnp.clip(p_hi - qt * NUM_OT, 0, NUM_OT)
        cnt_t = cnts[t]
        base_t = offs[t]
        nch = jnp.where(hi_t > lo_t,
                        jnp.maximum((cnt_t + CHUNK - 1) // CHUNK, 1), 0)

        def chunk_body(ch, c0, qt=qt, lo_t=lo_t, hi_t=hi_t,
                       cnt_t=cnt_t, base_t=base_t):
            gather_rows(base_t, ch)
            n = jnp.minimum(cnt_t - ch * CHUNK, CHUNK)
            nvec = jnp.maximum((n + 15) // 16, 0)

            for k in range(10):
                cnts_b[pl.ds(k * 16, 16)] = jnp.zeros((16,), jnp.int32)
                fill_b[pl.ds(k * 16, 16)] = jnp.zeros((16,), jnp.int32)

            for j in range(BOX):
                cj = jnp.full((16,), j, jnp.int32)

                def hist_body(k, c1, cj=cj):
                    lane = lax.iota(jnp.int32, 16) + k * 16
                    m = lane < n
                    o1 = plsc.load_gather(rows_c, [lane, cj], mask=m)
                    m2 = m & (o1 >= lo_t) & (o1 < hi_t)
                    plsc.addupdate_scatter(cnts_b, [o1 - lo_t], ones, mask=m2)
                    return c1
                lax.fori_loop(0, nvec, hist_body, 0)

            run = jnp.int32(0)
            for k in range(10):
                v = cnts_b[pl.ds(k * 16, 16)]
                incl = plsc.cumsum(v)
                bases_b[pl.ds(k * 16, 16)] = incl - v + run
                run = run + incl[15]

            for j in range(BOX):
                cj = jnp.full((16,), j, jnp.int32)

                def place_body(k, c1, cj=cj, j=j):
                    lane = lax.iota(jnp.int32, 16) + k * 16
                    m = lane < n
                    o1 = plsc.load_gather(rows_c, [lane, cj], mask=m)
                    m2 = m & (o1 >= lo_t) & (o1 < hi_t)
                    b = o1 - lo_t
                    cur = plsc.load_gather(fill_b, [b], mask=m2)
                    rank = plsc.scan_count(b, m2)[0] - 1
                    bv = plsc.load_gather(bases_b, [b], mask=m2)
                    pos = bv + cur + rank
                    plsc.store_scatter(entries, [pos], lane * 16 + j, mask=m2)
                    plsc.addupdate_scatter(fill_b, [b], ones, mask=m2)
                    return c1
                lax.fori_loop(0, nvec, place_body, 0)

            def apply_plane(o1, slb, lo_t=lo_t):
                bsel = jnp.full((16,), o1 - lo_t, jnp.int32)
                eb = plsc.load_gather(bases_b, [bsel])[0]
                ecnt = plsc.load_gather(cnts_b, [bsel])[0]

                def ent_body(kk, c2):
                    lanez = lax.iota(jnp.int32, 16) + kk * 16
                    m3 = lanez < ecnt
                    ent = plsc.load_gather(entries, [eb + lanez], mask=m3)
                    row = ent // 16
                    jv = ent - row * 16
                    attj = plsc.bitcast(
                        plsc.load_gather(rows_c, [row, jv + 10], mask=m3),
                        jnp.float32)
                    for i in range(BOX):
                        ci = jnp.full((16,), i, jnp.int32)
                        m4 = m3 & (jv != i)
                        o2 = plsc.load_gather(rows_c, [row, ci], mask=m4)
                        atti = plsc.bitcast(
                            plsc.load_gather(rows_c, [row, ci + 10], mask=m4),
                            jnp.float32)
                        rav = i * 9 + jnp.where(jv < i, jv, jv - 1)
                        plsc.addupdate_scatter(
                            slb, [o2, rav], attj * atti, mask=m4)
                    return c2

                lax.fori_loop(0, (ecnt + 15) // 16, ent_body, 0)

            def pair_body(pp, c1, qt=qt, lo_t=lo_t):
                o1 = lo_t + 2 * pp
                ha = pltpu.async_copy(score_hbm.at[qt, o1], slab, sem_a)
                hb = pltpu.async_copy(score_hbm.at[qt, o1 + 1], slab_b, sem_b)
                ha.wait()
                apply_plane(o1, slab)
                hb.wait()
                wa = pltpu.async_copy(slab, out_hbm.at[qt, o1], sem_a)
                apply_plane(o1 + 1, slab_b)
                wb = pltpu.async_copy(slab_b, out_hbm.at[qt, o1 + 1], sem_b)
                wa.wait()
                wb.wait()
                return c1

            lax.fori_loop(0, (hi_t - lo_t) // 2, pair_body, 0)

            @pl.when(((hi_t - lo_t) % 2 == 1) & (hi_t > lo_t))
            def _(qt=qt, hi_t=hi_t):
                o1 = hi_t - 1
                pltpu.sync_copy(score_hbm.at[qt, o1], slab)
                apply_plane(o1, slab)
                pltpu.sync_copy(slab, out_hbm.at[qt, o1])
            return c0

        lax.fori_loop(0, nch, chunk_body, 0)


@jax.jit
def kernel(obj_label, qus_type, attention, score_matrix):
    data = jnp.zeros((B, 128), jnp.int32)
    data = data.at[:, :BOX].set(obj_label)
    data = data.at[:, BOX:2 * BOX].set(
        lax.bitcast_convert_type(attention, jnp.int32))

    score_t = jnp.transpose(score_matrix, (0, 2, 3, 1))

    mesh = plsc.VectorSubcoreMesh(core_axis_name="c", subcore_axis_name="s")
    out_t = pl.kernel(
        _sc_body,
        out_type=jax.ShapeDtypeStruct(
            (NUM_QT, NUM_OT, NUM_OT, PAIR_NUM), jnp.float32),
        mesh=mesh,
        compiler_params=pltpu.CompilerParams(needs_layout_passes=False),
        scratch_types=[
            pltpu.VMEM((NUM_OT, PAIR_NUM), jnp.float32),
            pltpu.VMEM((NUM_OT, PAIR_NUM), jnp.float32),
            pltpu.VMEM((CHUNK, 128), jnp.int32),
            pltpu.VMEM((IDXN,), jnp.int32),
            pltpu.VMEM((8, 128), jnp.int32),
            pltpu.VMEM((B,), jnp.int32),
            pltpu.VMEM((ENTN,), jnp.int32),
            pltpu.VMEM((160,), jnp.int32),
            pltpu.VMEM((160,), jnp.int32),
            pltpu.VMEM((160,), jnp.int32),
            pltpu.SemaphoreType.DMA,
            pltpu.SemaphoreType.DMA,
            pltpu.SemaphoreType.DMA,
        ],
    )(score_t, data, qus_type)
    return jnp.transpose(out_t, (0, 3, 1, 2))


# speedup vs baseline: 10.8299x; 1.0105x over previous
import functools

import jax
import jax.numpy as jnp
from jax import lax
from jax.experimental import pallas as pl
from jax.experimental.pallas import tpu as pltpu
from jax.experimental.pallas import tpu_sc as plsc

NUM_QT = 65
NUM_OT = 151
PAIR_NUM = 90
BOX = 10
B = 16384
NPLANES = NUM_QT * NUM_OT
CHUNK = 384
NSUB = CHUNK // 128
IDXN = 17440
ENTN = CHUNK * BOX + 16

NC, NS = 2, 16
NW = NC * NS


def _sc_body(score_hbm, data_hbm, qtyp_hbm, out_hbm,
             slab, slab_b, rows_c, idx_buf, idx_stage, qtyp_v,
             entries, cnts_b, bases_b, fill_b, sem, sem_a, sem_b):
    w = lax.axis_index("s") * NC + lax.axis_index("c")
    p_lo = (w * NPLANES) // NW
    p_hi = ((w + 1) * NPLANES) // NW
    qt0 = p_lo // NUM_OT
    qtN = (p_hi - 1) // NUM_OT

    pltpu.sync_copy(qtyp_hbm, qtyp_v)

    def zero_body(k, c):
        idx_buf[pl.ds(k * 16, 16)] = jnp.zeros((16,), jnp.int32)
        return c
    lax.fori_loop(0, IDXN // 16, zero_body, 0)

    offs = []
    cnts = []
    cursor = jnp.int32(0)
    for t in range(4):
        qt_t = qt0 + t
        valid_t = qt_t <= qtN

        def scan_body(k, cur):
            ids = lax.iota(jnp.int32, 16) + k * 16
            v = qtyp_v[pl.ds(k * 16, 16)]
            m = (v == qt_t) & valid_t
            mi = m.astype(jnp.int32)
            incl = plsc.cumsum(mi)
            pos = cur + incl - mi
            plsc.store_scatter(idx_buf, [pos], ids, mask=m)
            return cur + incl[15]

        new_cursor = lax.fori_loop(0, B // 16, scan_body, cursor)
        offs.append(cursor)
        cnts.append(new_cursor - cursor)
        cursor = ((new_cursor + 15) // 16) * 16

    def gather_rows(basex, ch):
        def sub(d, c):
            def stage(k, c2):
                idx_stage[d, pl.ds(k * 16, 16)] = idx_buf[
                    pl.ds(basex + ch * CHUNK + d * 128 + k * 16, 16)]
                return c2
            lax.fori_loop(0, 8, stage, 0)
            pltpu.async_copy(data_hbm.at[idx_stage.at[d]],
                             rows_c.at[pl.ds(d * 128, 128)], sem).wait()
            return c
        lax.fori_loop(0, NSUB, sub, 0)

    ones = jnp.ones((16,), jnp.int32)

    for t in range(4):
        qt = qt0 + t
        lo_t = jnp.clip(p_lo - qt * NUM_OT, 0, NUM_OT)
        hi_t = jnp.clip(p_hi - qt * NUM_OT, 0, NUM_OT)
        cnt_t = cnts[t]
        base_t = offs[t]
        nch = jnp.where(hi_t > lo_t,
                        jnp.maximum((cnt_t + CHUNK - 1) // CHUNK, 1), 0)

        def chunk_body(ch, c0, qt=qt, lo_t=lo_t, hi_t=hi_t,
                       cnt_t=cnt_t, base_t=base_t):
            gather_rows(base_t, ch)
            n = jnp.minimum(cnt_t - ch * CHUNK, CHUNK)
            nvec = jnp.maximum((n + 15) // 16, 0)

            for k in range(10):
                cnts_b[pl.ds(k * 16, 16)] = jnp.zeros((16,), jnp.int32)
                fill_b[pl.ds(k * 16, 16)] = jnp.zeros((16,), jnp.int32)

            for j in range(BOX):
                cj = jnp.full((16,), j, jnp.int32)

                def hist_body(k, c1, cj=cj):
                    lane = lax.iota(jnp.int32, 16) + k * 16
                    m = lane < n
                    o1 = plsc.load_gather(rows_c, [lane, cj], mask=m)
                    m2 = m & (o1 >= lo_t) & (o1 < hi_t)
                    plsc.addupdate_scatter(cnts_b, [o1 - lo_t], ones, mask=m2)
                    return c1
                lax.fori_loop(0, nvec, hist_body, 0)

            run = jnp.int32(0)
            for k in range(10):
                v = cnts_b[pl.ds(k * 16, 16)]
                incl = plsc.cumsum(v)
                bases_b[pl.ds(k * 16, 16)] = incl - v + run
                run = run + incl[15]

            for j in range(BOX):
                cj = jnp.full((16,), j, jnp.int32)

                def place_body(k, c1, cj=cj, j=j):
                    lane = lax.iota(jnp.int32, 16) + k * 16
                    m = lane < n
                    o1 = plsc.load_gather(rows_c, [lane, cj], mask=m)
                    m2 = m & (o1 >= lo_t) & (o1 < hi_t)
                    b = o1 - lo_t
                    cur = plsc.load_gather(fill_b, [b], mask=m2)
                    rank = plsc.scan_count(b, m2)[0] - 1
                    bv = plsc.load_gather(bases_b, [b], mask=m2)
                    pos = bv + cur + rank
                    plsc.store_scatter(entries, [pos], lane * 16 + j, mask=m2)
                    plsc.addupdate_scatter(fill_b, [b], ones, mask=m2)
                    return c1
                lax.fori_loop(0, nvec, place_body, 0)

            def apply_plane(o1, slb, lo_t=lo_t):
                bsel = jnp.full((16,), o1 - lo_t, jnp.int32)
                eb = plsc.load_gather(bases_b, [bsel])[0]
                ecnt = plsc.load_gather(cnts_b, [bsel])[0]

                def ent_body(kk, c2):
                    lanez = lax.iota(jnp.int32, 16) + kk * 16
                    m3 = lanez < ecnt
                    ent = plsc.load_gather(entries, [eb + lanez], mask=m3)
                    row = ent // 16
                    jv = ent - row * 16
                    attj = plsc.bitcast(
                        plsc.load_gather(rows_c, [row, jv + 10], mask=m3),
                        jnp.float32)
                    for i in range(BOX):
                        ci = jnp.full((16,), i, jnp.int32)
                        m4 = m3 & (jv != i)
                        o2 = plsc.load_gather(rows_c, [row, ci], mask=m4)
                        atti = plsc.bitcast(
                            plsc.load_gather(rows_c, [row, ci + 10], mask=m4),
                            jnp.float32)
                        rav = i * 9 + jnp.where(jv < i, jv, jv - 1)
                        plsc.addupdate_scatter(
                            slb, [o2, rav], attj * atti, mask=m4)
                    return c2

                lax.fori_loop(0, (ecnt + 15) // 16, ent_body, 0)

            def pair_body(pp, c1, qt=qt, lo_t=lo_t, ch=ch):
                o1 = lo_t + 2 * pp

                @pl.when(ch == 0)
                def _():
                    ha = pltpu.async_copy(score_hbm.at[qt, o1], slab, sem_a)
                    hb = pltpu.async_copy(
                        score_hbm.at[qt, o1 + 1], slab_b, sem_b)
                    ha.wait()
                    hb.wait()

                @pl.when(ch > 0)
                def _():
                    ha = pltpu.async_copy(out_hbm.at[qt, o1], slab, sem_a)
                    hb = pltpu.async_copy(
                        out_hbm.at[qt, o1 + 1], slab_b, sem_b)
                    ha.wait()
                    hb.wait()

                apply_plane(o1, slab)
                wa = pltpu.async_copy(slab, out_hbm.at[qt, o1], sem_a)
                apply_plane(o1 + 1, slab_b)
                wb = pltpu.async_copy(slab_b, out_hbm.at[qt, o1 + 1], sem_b)
                wa.wait()
                wb.wait()
                return c1

            lax.fori_loop(0, (hi_t - lo_t) // 2, pair_body, 0)

            @pl.when(((hi_t - lo_t) % 2 == 1) & (hi_t > lo_t))
            def _(qt=qt, hi_t=hi_t, ch=ch):
                o1 = hi_t - 1

                @pl.when(ch == 0)
                def _():
                    pltpu.sync_copy(score_hbm.at[qt, o1], slab)

                @pl.when(ch > 0)
                def _():
                    pltpu.sync_copy(out_hbm.at[qt, o1], slab)

                apply_plane(o1, slab)
                pltpu.sync_copy(slab, out_hbm.at[qt, o1])
            return c0

        lax.fori_loop(0, nch, chunk_body, 0)


@jax.jit
def kernel(obj_label, qus_type, attention, score_matrix):
    data = jnp.zeros((B, 128), jnp.int32)
    data = data.at[:, :BOX].set(obj_label)
    data = data.at[:, BOX:2 * BOX].set(
        lax.bitcast_convert_type(attention, jnp.int32))

    score_t = jnp.transpose(score_matrix, (0, 2, 3, 1))

    mesh = plsc.VectorSubcoreMesh(core_axis_name="c", subcore_axis_name="s")
    out_t = pl.kernel(
        _sc_body,
        out_type=jax.ShapeDtypeStruct(
            (NUM_QT, NUM_OT, NUM_OT, PAIR_NUM), jnp.float32),
        mesh=mesh,
        compiler_params=pltpu.CompilerParams(needs_layout_passes=False),
        scratch_types=[
            pltpu.VMEM((NUM_OT, PAIR_NUM), jnp.float32),
            pltpu.VMEM((NUM_OT, PAIR_NUM), jnp.float32),
            pltpu.VMEM((CHUNK, 128), jnp.int32),
            pltpu.VMEM((IDXN,), jnp.int32),
            pltpu.VMEM((8, 128), jnp.int32),
            pltpu.VMEM((B,), jnp.int32),
            pltpu.VMEM((ENTN,), jnp.int32),
            pltpu.VMEM((160,), jnp.int32),
            pltpu.VMEM((160,), jnp.int32),
            pltpu.VMEM((160,), jnp.int32),
            pltpu.SemaphoreType.DMA,
            pltpu.SemaphoreType.DMA,
            pltpu.SemaphoreType.DMA,
        ],
    )(score_t, data, qus_type)
    return jnp.transpose(out_t, (0, 3, 1, 2))
